# Initial kernel scaffold; baseline (speedup 1.0000x reference)
import jax
import jax.numpy as jnp
from jax.experimental import pallas as pl


def kernel(x, edge_index, batch, W0, b0, gamma0, beta0, W1, b1, gamma1, beta1, W2, b2, gamma2, beta2):
    raise NotImplementedError("write your pallas kernel here")


# speedup vs baseline: 14.6639x; 14.6639x over previous
import functools

import jax
import jax.numpy as jnp
from jax import lax
from jax.experimental import pallas as pl
from jax.experimental.pallas import tpu as pltpu
from jax.experimental.pallas import tpu_sc as plsc

N = 10000
E = 320000
G = 64
D = 128
BN_EPS = 1e-4
SLOPE = 0.01

NC = 2
NS = 16
NW = NC * NS
EPT = E // NW
CH = 80
NCHUNK = EPT // CH
NP = 10240
RPT = NP // NS
ZR = 128

R_BLK = 2000


def _mesh():
    return plsc.VectorSubcoreMesh(core_axis_name="c", subcore_axis_name="s")


@functools.partial(
    pl.kernel,
    out_type=jax.ShapeDtypeStruct((NC, NP, D), jnp.float32),
    mesh=_mesh(),
    scratch_types=[
        pltpu.VMEM((NCHUNK, CH), jnp.int32),
        pltpu.VMEM((CH, D), jnp.float32),
        pltpu.VMEM_SHARED((NP, D), jnp.float32),
    ],
)
def _sc_degree(dst_hbm, ones_hbm, zeros_hbm, out_hbm, di_v, ones_v, acc_sh):
    cid = lax.axis_index("c")
    sid = lax.axis_index("s")
    wid = sid * NC + cid
    row0 = sid * RPT

    for k in range(RPT // ZR):
        pltpu.sync_copy(zeros_hbm, acc_sh.at[pl.ds(row0 + k * ZR, ZR)])
    pltpu.sync_copy(ones_hbm, ones_v)
    pltpu.sync_copy(dst_hbm.at[wid], di_v)

    plsc.subcore_barrier()

    def _edges(j, c):
        pltpu.sync_copy(ones_v, acc_sh.at[di_v.at[j]], add=True)
        return c

    lax.fori_loop(0, NCHUNK, _edges, 0)

    plsc.subcore_barrier()

    for k in range(RPT // ZR):
        r = row0 + k * ZR
        pltpu.sync_copy(acc_sh.at[pl.ds(r, ZR)], out_hbm.at[cid, pl.ds(r, ZR)])


@functools.partial(
    pl.kernel,
    out_type=jax.ShapeDtypeStruct((NC, NP, D), jnp.float32),
    mesh=_mesh(),
    scratch_types=[
        pltpu.VMEM((NCHUNK, CH), jnp.int32),
        pltpu.VMEM((NCHUNK, CH), jnp.int32),
        pltpu.VMEM((CH, D), jnp.float32),
        pltpu.VMEM_SHARED((NP, D), jnp.float32),
        pltpu.SemaphoreType.DMA,
    ],
)
def _sc_scatter(src_hbm, dst_hbm, hp_hbm, zeros_hbm, out_hbm,
                si_v, di_v, buf, acc_sh, sem):
    cid = lax.axis_index("c")
    sid = lax.axis_index("s")
    wid = sid * NC + cid
    row0 = sid * RPT

    for k in range(RPT // ZR):
        pltpu.sync_copy(zeros_hbm, acc_sh.at[pl.ds(row0 + k * ZR, ZR)])
    pltpu.sync_copy(src_hbm.at[wid], si_v)
    pltpu.sync_copy(dst_hbm.at[wid], di_v)

    plsc.subcore_barrier()

    def _edges(j, c):
        pltpu.async_copy(hp_hbm.at[si_v.at[j]], buf, sem).wait()
        pltpu.sync_copy(buf, acc_sh.at[di_v.at[j]], add=True)
        return c

    lax.fori_loop(0, NCHUNK, _edges, 0)

    plsc.subcore_barrier()

    for k in range(RPT // ZR):
        r = row0 + k * ZR
        pltpu.sync_copy(acc_sh.at[pl.ds(r, ZR)], out_hbm.at[cid, pl.ds(r, ZR)])


def _tc_dis(cnt):
    def body(cnt_ref, dis_ref):
        deg = cnt_ref[0, :, 0:1] + cnt_ref[1, :, 0:1] + 1.0
        dis_ref[...] = lax.rsqrt(deg)

    return pl.pallas_call(
        body,
        out_shape=jax.ShapeDtypeStruct((NP, 1), jnp.float32),
    )(cnt)


def _tc_matmul(inp, W, dis_col, s, t):
    def body(x_ref, w_ref, d_ref, s_ref, t_ref, o_ref):
        xb = (x_ref[...] * s_ref[...] + t_ref[...]) * d_ref[...]
        o_ref[...] = jnp.dot(xb, w_ref[...], preferred_element_type=jnp.float32)

    return pl.pallas_call(
        body,
        grid=(N // R_BLK,),
        in_specs=[
            pl.BlockSpec((R_BLK, D), lambda i: (i, 0)),
            pl.BlockSpec((D, D), lambda i: (0, 0)),
            pl.BlockSpec((R_BLK, 1), lambda i: (i, 0)),
            pl.BlockSpec((1, D), lambda i: (0, 0)),
            pl.BlockSpec((1, D), lambda i: (0, 0)),
        ],
        out_specs=pl.BlockSpec((R_BLK, D), lambda i: (i, 0)),
        out_shape=jax.ShapeDtypeStruct((N, D), jnp.float32),
    )(inp, W, dis_col, s, t)


def _tc_post(p, hp, dis_col, b, batch_col):
    def body(p_ref, h_ref, d_ref, b_ref, bt_ref, a_ref, st_ref, seg_ref, c_ref):
        i = pl.program_id(0)
        tot = p_ref[0] + p_ref[1] + h_ref[...]
        conv = tot * d_ref[...] + b_ref[...]
        a = jnp.where(conv >= 0, conv, SLOPE * conv)
        a_ref[...] = a
        ssum = jnp.sum(a, axis=0, keepdims=True)
        ssq = jnp.sum(a * a, axis=0, keepdims=True)
        st = jnp.concatenate([ssum, ssq], axis=0)
        oh = (bt_ref[...] == lax.broadcasted_iota(jnp.int32, (R_BLK, G), 1)
              ).astype(jnp.float32)
        seg = lax.dot_general(oh, a, (((0,), (0,)), ((), ())),
                              preferred_element_type=jnp.float32)
        cnt = jnp.sum(oh, axis=0, keepdims=True)

        @pl.when(i == 0)
        def _():
            st_ref[...] = st
            seg_ref[...] = seg
            c_ref[...] = cnt

        @pl.when(i != 0)
        def _():
            st_ref[...] += st
            seg_ref[...] += seg
            c_ref[...] += cnt

    return pl.pallas_call(
        body,
        grid=(N // R_BLK,),
        in_specs=[
            pl.BlockSpec((NC, R_BLK, D), lambda i: (0, i, 0)),
            pl.BlockSpec((R_BLK, D), lambda i: (i, 0)),
            pl.BlockSpec((R_BLK, 1), lambda i: (i, 0)),
            pl.BlockSpec((1, D), lambda i: (0, 0)),
            pl.BlockSpec((R_BLK, 1), lambda i: (i, 0)),
        ],
        out_specs=[
            pl.BlockSpec((R_BLK, D), lambda i: (i, 0)),
            pl.BlockSpec((2, D), lambda i: (0, 0)),
            pl.BlockSpec((G, D), lambda i: (0, 0)),
            pl.BlockSpec((1, G), lambda i: (0, 0)),
        ],
        out_shape=[
            jax.ShapeDtypeStruct((N, D), jnp.float32),
            jax.ShapeDtypeStruct((2, D), jnp.float32),
            jax.ShapeDtypeStruct((G, D), jnp.float32),
            jax.ShapeDtypeStruct((1, G), jnp.float32),
        ],
    )(p, hp, dis_col, b, batch_col)


def _tc_affine(st, gamma, beta):
    def body(st_ref, g_ref, be_ref, s_ref, t_ref):
        mean = st_ref[0:1, :] * (1.0 / N)
        var = st_ref[1:2, :] * (1.0 / N) - mean * mean
        s = g_ref[...] * lax.rsqrt(var + BN_EPS)
        s_ref[...] = s
        t_ref[...] = be_ref[...] - mean * s

    return pl.pallas_call(
        body,
        out_shape=[
            jax.ShapeDtypeStruct((1, D), jnp.float32),
            jax.ShapeDtypeStruct((1, D), jnp.float32),
        ],
    )(st, gamma, beta)


def _tc_pool(seg0, seg1, seg2, s0, t0, s1, t1, s2, t2, cnt_col):
    def body(g0, g1, g2, s0r, t0r, s1r, t1r, s2r, t2r, c_ref, o_ref):
        c = c_ref[...]
        cm = jnp.maximum(c, 1.0)
        for l, (gr, sr, tr) in enumerate(((g0, s0r, t0r), (g1, s1r, t1r),
                                          (g2, s2r, t2r))):
            pool = gr[...] / cm
            val = jnp.where(c > 0, pool * sr[...] + tr[...], 0.0)
            o_ref[:, l * D:(l + 1) * D] = val

    return pl.pallas_call(
        body,
        out_shape=jax.ShapeDtypeStruct((G, 3 * D), jnp.float32),
    )(seg0, seg1, seg2, s0, t0, s1, t1, s2, t2, cnt_col)


def _tc_bn(a, s, t):
    def body(a_ref, s_ref, t_ref, o_ref):
        o_ref[...] = a_ref[...] * s_ref[...] + t_ref[...]

    return pl.pallas_call(
        body,
        grid=(N // R_BLK,),
        in_specs=[
            pl.BlockSpec((R_BLK, D), lambda i: (i, 0)),
            pl.BlockSpec((1, D), lambda i: (0, 0)),
            pl.BlockSpec((1, D), lambda i: (0, 0)),
        ],
        out_specs=pl.BlockSpec((R_BLK, D), lambda i: (i, 0)),
        out_shape=jax.ShapeDtypeStruct((N, D), jnp.float32),
    )(a, s, t)


def kernel(x, edge_index, batch,
           W0, b0, gamma0, beta0,
           W1, b1, gamma1, beta1,
           W2, b2, gamma2, beta2):
    src = edge_index[0].reshape(NW, NCHUNK, CH)
    dst = edge_index[1].reshape(NW, NCHUNK, CH)

    ones_cd = jnp.ones((CH, D), jnp.float32)
    zeros_d = jnp.zeros((ZR, D), jnp.float32)

    cnt = _sc_degree(dst, ones_cd, zeros_d)
    dis_col = _tc_dis(cnt)
    batch_col = batch.reshape(N, 1)

    ones = jnp.ones((1, D), jnp.float32)
    zeros = jnp.zeros((1, D), jnp.float32)
    params = [(W0, b0, gamma0, beta0), (W1, b1, gamma1, beta1),
              (W2, b2, gamma2, beta2)]

    inp = x
    s_prev, t_prev = ones, zeros
    segs, ss, ts = [], [], []
    cnt_g = None
    for (W, b, g, be) in params:
        hp = _tc_matmul(inp, W, dis_col, s_prev, t_prev)
        p = _sc_scatter(src, dst, hp, zeros_d)
        a, st, seg, c = _tc_post(p, hp, dis_col, b.reshape(1, D), batch_col)
        s_l, t_l = _tc_affine(st, g.reshape(1, D), be.reshape(1, D))
        segs.append(seg)
        ss.append(s_l)
        ts.append(t_l)
        cnt_g = c
        inp = a
        s_prev, t_prev = s_l, t_l

    xpool = _tc_pool(segs[0], segs[1], segs[2],
                     ss[0], ts[0], ss[1], ts[1], ss[2], ts[2],
                     cnt_g.reshape(G, 1))
    bn2 = _tc_bn(inp, ss[2], ts[2])
    return (xpool, bn2)
